# baseline (speedup 1.0000x reference)
import functools
import math

import jax
import jax.numpy as jnp
from jax.experimental import pallas as pl
from jax.experimental.pallas import tpu as pltpu

D_MODEL = 1024
N_EXPERTS = 8
TOP_K = 2
D_FF = 4 * D_MODEL

BLK = 256
_SQRT_HALF = 1.0 / math.sqrt(2.0)


def _ffn_body(gid_ref, xs_ref, w1_ref, b1_ref, w2_ref, b2_ref, out_ref):
    xs = xs_ref[...]
    h = jnp.dot(xs, w1_ref[0], preferred_element_type=jnp.float32)
    h = h + b1_ref[0, 0].astype(jnp.float32)
    h = 0.5 * h * (1.0 + jax.lax.erf(h * _SQRT_HALF))
    y = jnp.dot(h.astype(w2_ref.dtype), w2_ref[0],
                preferred_element_type=jnp.float32)
    out_ref[...] = y + b2_ref[0, 0].astype(jnp.float32)


def _grouped_ffn(gid, xs, W1, b1, W2, b2, n_blocks, interpret=False):
    grid_spec = pltpu.PrefetchScalarGridSpec(
        num_scalar_prefetch=1,
        grid=(n_blocks,),
        in_specs=[
            pl.BlockSpec((BLK, D_MODEL), lambda i, gid: (i, 0)),
            pl.BlockSpec((1, D_MODEL, D_FF), lambda i, gid: (gid[i], 0, 0)),
            pl.BlockSpec((1, 1, D_FF), lambda i, gid: (gid[i], 0, 0)),
            pl.BlockSpec((1, D_FF, D_MODEL), lambda i, gid: (gid[i], 0, 0)),
            pl.BlockSpec((1, 1, D_MODEL), lambda i, gid: (gid[i], 0, 0)),
        ],
        out_specs=pl.BlockSpec((BLK, D_MODEL), lambda i, gid: (i, 0)),
    )
    return pl.pallas_call(
        _ffn_body,
        grid_spec=grid_spec,
        out_shape=jax.ShapeDtypeStruct((n_blocks * BLK, D_MODEL), jnp.float32),
        compiler_params=pltpu.CompilerParams(
            dimension_semantics=("arbitrary",)),
        interpret=interpret,
    )(gid, xs, W1, b1, W2, b2)


def kernel(x, Wr, br, W1, b1, W2, b2, interpret=False):
    B, L, D = x.shape
    xf = x.reshape(-1, D)
    N = xf.shape[0]
    A = N * TOP_K

    logits = xf @ Wr + br
    rw = jax.nn.softmax(logits, axis=-1)
    tkw, tki = jax.lax.top_k(rw, TOP_K)
    tkw = tkw / jnp.sum(tkw, axis=-1, keepdims=True)

    n_blocks = (A + N_EXPERTS * (BLK - 1) + BLK - 1) // BLK
    R = n_blocks * BLK

    e_flat = tki.reshape(-1).astype(jnp.int32)
    oh = (e_flat[:, None] == jnp.arange(N_EXPERTS, dtype=jnp.int32)[None, :]
          ).astype(jnp.int32)
    incl = jnp.cumsum(oh, axis=0)
    rank = jnp.take_along_axis(incl, e_flat[:, None], axis=1)[:, 0] - 1
    counts = incl[-1]
    pc = ((counts + BLK - 1) // BLK) * BLK
    cum_pc = jnp.cumsum(pc)
    pad_off = cum_pc - pc
    dest = pad_off[e_flat] + rank
    tok = jnp.arange(A, dtype=jnp.int32) // TOP_K

    xs = jnp.zeros((R, D), x.dtype).at[dest].set(xf[tok])
    pos = dest.reshape(N, TOP_K)
    gid = jnp.minimum(
        jnp.searchsorted(cum_pc, jnp.arange(n_blocks, dtype=jnp.int32) * BLK,
                         side='right'),
        N_EXPERTS - 1).astype(jnp.int32)

    bf = jnp.bfloat16
    ys = _grouped_ffn(gid, xs.astype(bf), W1.astype(bf),
                      b1.reshape(N_EXPERTS, 1, D_FF), W2.astype(bf),
                      b2.reshape(N_EXPERTS, 1, D_MODEL), n_blocks,
                      interpret=interpret)

    out = ys[pos[:, 0]] * tkw[:, :1] + ys[pos[:, 1]] * tkw[:, 1:]
    return out.reshape(B, L, D)


# speedup vs baseline: 1.0286x; 1.0286x over previous
import functools
import math

import jax
import jax.numpy as jnp
from jax.experimental import pallas as pl
from jax.experimental.pallas import tpu as pltpu
from jax.experimental.pallas import tpu_sc as plsc

D_MODEL = 1024
N_EXPERTS = 8
TOP_K = 2
D_FF = 4 * D_MODEL

BLK = 256
_SQRT_HALF = 1.0 / math.sqrt(2.0)


def _ffn_body(gid_ref, xs_ref, w1_ref, b1_ref, w2_ref, b2_ref, out_ref):
    xs = xs_ref[...]
    h = jnp.dot(xs, w1_ref[0], preferred_element_type=jnp.float32)
    h = h + b1_ref[0, 0].astype(jnp.float32)
    h = 0.5 * h * (1.0 + jax.lax.erf(h * _SQRT_HALF))
    y = jnp.dot(h.astype(w2_ref.dtype), w2_ref[0],
                preferred_element_type=jnp.float32)
    out_ref[...] = y + b2_ref[0, 0].astype(jnp.float32)


def _grouped_ffn(gid, xs, W1, b1, W2, b2, n_blocks, interpret=False):
    grid_spec = pltpu.PrefetchScalarGridSpec(
        num_scalar_prefetch=1,
        grid=(n_blocks,),
        in_specs=[
            pl.BlockSpec((BLK, D_MODEL), lambda i, gid: (i, 0)),
            pl.BlockSpec((1, D_MODEL, D_FF), lambda i, gid: (gid[i], 0, 0)),
            pl.BlockSpec((1, 1, D_FF), lambda i, gid: (gid[i], 0, 0)),
            pl.BlockSpec((1, D_FF, D_MODEL), lambda i, gid: (gid[i], 0, 0)),
            pl.BlockSpec((1, 1, D_MODEL), lambda i, gid: (gid[i], 0, 0)),
        ],
        out_specs=pl.BlockSpec((BLK, D_MODEL), lambda i, gid: (i, 0)),
    )
    return pl.pallas_call(
        _ffn_body,
        grid_spec=grid_spec,
        out_shape=jax.ShapeDtypeStruct((n_blocks * BLK, D_MODEL), jnp.float32),
        compiler_params=pltpu.CompilerParams(
            dimension_semantics=("arbitrary",)),
        interpret=interpret,
    )(gid, xs, W1, b1, W2, b2)


_SC_WIN = 128
_SC_SUB = 512


def _sc_dispatch(xf, tok, dest, R):
    A2 = dest.shape[1]
    xf2 = xf.reshape(-1, _SC_SUB)
    mesh = plsc.VectorSubcoreMesh(core_axis_name="core",
                                  subcore_axis_name="subcore")

    @functools.partial(
        pl.kernel,
        out_type=jax.ShapeDtypeStruct((R * (D_MODEL // _SC_SUB), _SC_SUB),
                                      xf.dtype),
        mesh=mesh,
        scratch_types=[pltpu.VMEM((_SC_WIN, _SC_SUB), xf.dtype)])
    def k(x_hbm, tok_hbm, dest_hbm, xs_hbm, buf):
        def body(tok_vmem, dest_vmem):
            pltpu.sync_copy(x_hbm.at[tok_vmem.at[0]], buf)
            pltpu.sync_copy(buf, xs_hbm.at[dest_vmem.at[0]])

        pltpu.emit_pipeline(
            body,
            grid=(A2 // _SC_WIN,),
            in_specs=[pl.BlockSpec((1, _SC_WIN), lambda i: (0, i)),
                      pl.BlockSpec((1, _SC_WIN), lambda i: (0, i))],
            out_specs=[],
            core_axis_name=('core', 'subcore'),
            dimension_semantics=(pltpu.PARALLEL,),
        )(tok_hbm, dest_hbm)

    return k(xf2, tok, dest).reshape(R, D_MODEL)


def kernel(x, Wr, br, W1, b1, W2, b2, interpret=False):
    B, L, D = x.shape
    xf = x.reshape(-1, D)
    N = xf.shape[0]
    A = N * TOP_K

    logits = xf @ Wr + br
    rw = jax.nn.softmax(logits, axis=-1)
    tkw, tki = jax.lax.top_k(rw, TOP_K)
    tkw = tkw / jnp.sum(tkw, axis=-1, keepdims=True)

    n_blocks = (A + N_EXPERTS * (BLK - 1) + BLK - 1) // BLK
    R = n_blocks * BLK

    e_flat = tki.reshape(-1).astype(jnp.int32)
    oh = (e_flat[:, None] == jnp.arange(N_EXPERTS, dtype=jnp.int32)[None, :]
          ).astype(jnp.int32)
    incl = jnp.cumsum(oh, axis=0)
    rank = jnp.take_along_axis(incl, e_flat[:, None], axis=1)[:, 0] - 1
    counts = incl[-1]
    pc = ((counts + BLK - 1) // BLK) * BLK
    cum_pc = jnp.cumsum(pc)
    pad_off = cum_pc - pc
    dest = pad_off[e_flat] + rank
    tok = jnp.arange(A, dtype=jnp.int32) // TOP_K

    if interpret:
        xs = jnp.zeros((R, D), x.dtype).at[dest].set(xf[tok])
    else:
        nsub = D // _SC_SUB
        sub = jnp.arange(nsub, dtype=jnp.int32)
        tok2 = (tok[:, None] * nsub + sub).reshape(1, A * nsub)
        dest2 = (dest[:, None] * nsub + sub).reshape(1, A * nsub)
        xs = _sc_dispatch(xf, tok2, dest2, R)
    pos = dest.reshape(N, TOP_K)
    gid = jnp.minimum(
        jnp.searchsorted(cum_pc, jnp.arange(n_blocks, dtype=jnp.int32) * BLK,
                         side='right'),
        N_EXPERTS - 1).astype(jnp.int32)

    bf = jnp.bfloat16
    ys = _grouped_ffn(gid, xs.astype(bf), W1.astype(bf),
                      b1.reshape(N_EXPERTS, 1, D_FF), W2.astype(bf),
                      b2.reshape(N_EXPERTS, 1, D_MODEL), n_blocks,
                      interpret=interpret)

    out = ys[pos[:, 0]] * tkw[:, :1] + ys[pos[:, 1]] * tkw[:, 1:]
    return out.reshape(B, L, D)
